# baseline (speedup 1.0000x reference)
import functools
import math

import jax
import jax.numpy as jnp
from jax import lax
from jax.experimental import pallas as pl
from jax.experimental.pallas import tpu as pltpu
from jax.experimental.pallas import tpu_sc as plsc

F32 = jnp.float32
I32 = jnp.int32


def _silu(x):
    return x * (0.5 * jnp.tanh(0.5 * x) + 0.5)


def _sc_gather(table, idx):
    V, D = table.shape
    N = idx.shape[0]
    info = plsc.get_sparse_core_info()
    nw = info.num_cores * info.num_subcores
    b_per_w = N // nw
    mesh = plsc.VectorSubcoreMesh(core_axis_name="c", subcore_axis_name="s")

    @functools.partial(
        pl.kernel,
        out_type=jax.ShapeDtypeStruct((N, D), F32),
        mesh=mesh,
        scratch_types=[
            pltpu.VMEM((b_per_w,), I32),
            pltpu.VMEM((b_per_w, D), F32),
            pltpu.SemaphoreType.DMA,
        ],
    )
    def k(table_hbm, idx_hbm, out_hbm, idx_v, rows_v, sem):
        wid = lax.axis_index("s") * info.num_cores + lax.axis_index("c")
        base = wid * b_per_w
        pltpu.sync_copy(idx_hbm.at[pl.ds(base, b_per_w)], idx_v)
        pltpu.async_copy(table_hbm.at[idx_v], rows_v, sem).wait()
        pltpu.sync_copy(rows_v, out_hbm.at[pl.ds(base, b_per_w)])

    return k(table, idx)


def _add_pos(emb, pos):
    B, L, D = emb.shape

    def body(e_ref, p_ref, o_ref):
        o_ref[...] = e_ref[...] + p_ref[...][None]

    return pl.pallas_call(
        body,
        grid=(B,),
        in_specs=[
            pl.BlockSpec((1, L, D), lambda b: (b, 0, 0)),
            pl.BlockSpec((L, D), lambda b: (0, 0)),
        ],
        out_specs=pl.BlockSpec((1, L, D), lambda b: (b, 0, 0)),
        out_shape=jax.ShapeDtypeStruct((B, L, D), F32),
    )(emb, pos)


def _mm_split(h, w, di, tm=512):
    B, L, D = h.shape
    M = w.shape[0]

    def body(x_ref, w_ref, xc_ref, z_ref):
        o = lax.dot_general(x_ref[0], w_ref[...], (((1,), (1,)), ((), ())),
                            preferred_element_type=F32)
        xc_ref[0] = o[:, :di]
        z_ref[0] = o[:, di:]

    return pl.pallas_call(
        body,
        grid=(B, L // tm),
        in_specs=[
            pl.BlockSpec((1, tm, D), lambda b, i: (b, i, 0)),
            pl.BlockSpec((M, D), lambda b, i: (0, 0)),
        ],
        out_specs=[
            pl.BlockSpec((1, tm, di), lambda b, i: (b, i, 0)),
            pl.BlockSpec((1, tm, di), lambda b, i: (b, i, 0)),
        ],
        out_shape=[
            jax.ShapeDtypeStruct((B, L, di), F32),
            jax.ShapeDtypeStruct((B, L, di), F32),
        ],
    )(h, w)


def _mamba_core(xc, z, wt, cb, xw, dtw, dtb, at, dv, dtr, ds, ch=512):
    B, L, DI = xc.shape
    DC = wt.shape[0]

    def body(x_ref, z_ref, wt_ref, cb_ref, xw_ref, dtw_ref, dtb_ref, at_ref,
             dv_ref, o_ref, h_ref, tail_ref, sdt_ref, sdtx_ref, sbm_ref,
             scm_ref):
        c = pl.program_id(1)

        @pl.when(c == 0)
        def _():
            h_ref[...] = jnp.zeros_like(h_ref)
            tail_ref[...] = jnp.zeros_like(tail_ref)

        x = x_ref[0]
        tail = tail_ref[...]
        xp = jnp.concatenate([tail[8 - (DC - 1):], x], axis=0)
        acc = cb_ref[...]
        for k in range(DC):
            acc = acc + xp[k:k + ch] * wt_ref[k][None]
        xcv = _silu(acc)
        tail_ref[...] = x[ch - 8:]

        xdbl = lax.dot_general(xcv, xw_ref[...], (((1,), (1,)), ((), ())),
                               preferred_element_type=F32)
        dtp = lax.dot_general(xdbl[:, :dtr], dtw_ref[...],
                              (((1,), (1,)), ((), ())),
                              preferred_element_type=F32) + dtb_ref[...]
        dt = jnp.maximum(dtp, 0.0) + jnp.log(1.0 + jnp.exp(-jnp.abs(dtp)))
        sdt_ref[...] = dt
        sdtx_ref[...] = dt * xcv
        sbm_ref[...] = xdbl[:, dtr:dtr + ds]
        scm_ref[...] = xdbl[:, dtr + ds:dtr + 2 * ds]

        atv = at_ref[...]

        def step(t, h):
            dtt = sdt_ref[pl.ds(t, 1), :]
            da = jnp.exp(atv * dtt)
            btc = sbm_ref[pl.ds(t, 1), :][0][:, None]
            dtxt = sdtx_ref[pl.ds(t, 1), :]
            h = da * h + btc * dtxt
            ctc = scm_ref[pl.ds(t, 1), :][0][:, None]
            o_ref[0, pl.ds(t, 1), :] = jnp.sum(h * ctc, axis=0, keepdims=True)
            return h

        h_ref[...] = lax.fori_loop(0, ch, step, h_ref[...], unroll=8)
        o_ref[0] = (o_ref[0] + dv_ref[...] * xcv) * _silu(z_ref[0])

    nxp = xw.shape[0]
    return pl.pallas_call(
        body,
        grid=(B, L // ch),
        in_specs=[
            pl.BlockSpec((1, ch, DI), lambda b, c: (b, c, 0)),
            pl.BlockSpec((1, ch, DI), lambda b, c: (b, c, 0)),
            pl.BlockSpec((DC, DI), lambda b, c: (0, 0)),
            pl.BlockSpec((1, DI), lambda b, c: (0, 0)),
            pl.BlockSpec((nxp, DI), lambda b, c: (0, 0)),
            pl.BlockSpec((DI, dtr), lambda b, c: (0, 0)),
            pl.BlockSpec((1, DI), lambda b, c: (0, 0)),
            pl.BlockSpec((ds, DI), lambda b, c: (0, 0)),
            pl.BlockSpec((1, DI), lambda b, c: (0, 0)),
        ],
        out_specs=pl.BlockSpec((1, ch, DI), lambda b, c: (b, c, 0)),
        out_shape=jax.ShapeDtypeStruct((B, L, DI), F32),
        scratch_shapes=[
            pltpu.VMEM((ds, DI), F32),
            pltpu.VMEM((8, DI), F32),
            pltpu.VMEM((ch, DI), F32),
            pltpu.VMEM((ch, DI), F32),
            pltpu.VMEM((ch, ds), F32),
            pltpu.VMEM((ch, ds), F32),
        ],
    )(xc, z, wt, cb, xw, dtw, dtb, at, dv)


def _mm_ln_res(x, w, g, b, res, tm=512, eps=1e-5):
    N, K = x.shape
    M = w.shape[0]

    def body(x_ref, w_ref, g_ref, b_ref, r_ref, o_ref):
        o = lax.dot_general(x_ref[...], w_ref[...], (((1,), (1,)), ((), ())),
                            preferred_element_type=F32)
        mu = jnp.mean(o, axis=1, keepdims=True)
        d = o - mu
        var = jnp.mean(d * d, axis=1, keepdims=True)
        ln = d * lax.rsqrt(var + eps) * g_ref[...] + b_ref[...]
        o_ref[...] = r_ref[...] + ln

    return pl.pallas_call(
        body,
        grid=(N // tm,),
        in_specs=[
            pl.BlockSpec((tm, K), lambda i: (i, 0)),
            pl.BlockSpec((M, K), lambda i: (0, 0)),
            pl.BlockSpec((1, M), lambda i: (0, 0)),
            pl.BlockSpec((1, M), lambda i: (0, 0)),
            pl.BlockSpec((tm, M), lambda i: (i, 0)),
        ],
        out_specs=pl.BlockSpec((tm, M), lambda i: (i, 0)),
        out_shape=jax.ShapeDtypeStruct((N, M), F32),
    )(x, w, g, b, res)


def _scale_ln_res(x, tw, g, b, res, tm=512, eps=1e-5):
    N, M = x.shape

    def body(x_ref, t_ref, g_ref, b_ref, r_ref, o_ref):
        v = x_ref[...] * t_ref[...]
        mu = jnp.mean(v, axis=1, keepdims=True)
        d = v - mu
        var = jnp.mean(d * d, axis=1, keepdims=True)
        ln = d * lax.rsqrt(var + eps) * g_ref[...] + b_ref[...]
        o_ref[...] = r_ref[...] + ln

    return pl.pallas_call(
        body,
        grid=(N // tm,),
        in_specs=[
            pl.BlockSpec((tm, M), lambda i: (i, 0)),
            pl.BlockSpec((tm, 1), lambda i: (i, 0)),
            pl.BlockSpec((1, M), lambda i: (0, 0)),
            pl.BlockSpec((1, M), lambda i: (0, 0)),
            pl.BlockSpec((tm, M), lambda i: (i, 0)),
        ],
        out_specs=pl.BlockSpec((tm, M), lambda i: (i, 0)),
        out_shape=jax.ShapeDtypeStruct((N, M), F32),
    )(x, tw, g, b, res)


def _gate_top1(x, gw, gb, ne, tm=512):
    N, K = x.shape

    def body(x_ref, w_ref, b_ref, tw_ref, ti_ref, cnt_ref):
        i = pl.program_id(0)
        logits = lax.dot_general(x_ref[...], w_ref[...], (((1,), (1,)), ((), ())),
                                 preferred_element_type=F32) + b_ref[...]
        m = jnp.max(logits, axis=1, keepdims=True)
        e = jnp.exp(logits - m)
        s = jnp.sum(e, axis=1, keepdims=True)
        gwm = e / s
        top = jnp.max(gwm, axis=1, keepdims=True)
        lanes = lax.broadcasted_iota(I32, (tm, ne), 1)
        idx = jnp.min(jnp.where(gwm >= top, lanes, ne), axis=1, keepdims=True)
        tw_ref[...] = top
        ti_ref[...] = idx

        @pl.when(i == 0)
        def _():
            cnt_ref[...] = jnp.zeros_like(cnt_ref)

        eq = (idx == lax.broadcasted_iota(I32, (tm, ne), 1)).astype(I32)
        cnt_ref[...] += jnp.sum(eq, axis=0, keepdims=True)

    return pl.pallas_call(
        body,
        grid=(N // tm,),
        in_specs=[
            pl.BlockSpec((tm, K), lambda i: (i, 0)),
            pl.BlockSpec((ne, K), lambda i: (0, 0)),
            pl.BlockSpec((1, ne), lambda i: (0, 0)),
        ],
        out_specs=[
            pl.BlockSpec((tm, 1), lambda i: (i, 0)),
            pl.BlockSpec((tm, 1), lambda i: (i, 0)),
            pl.BlockSpec((1, ne), lambda i: (0, 0)),
        ],
        out_shape=[
            jax.ShapeDtypeStruct((N, 1), F32),
            jax.ShapeDtypeStruct((N, 1), I32),
            jax.ShapeDtypeStruct((1, ne), I32),
        ],
    )(x, gw, gb)


def _moe_ffn(xs, w1, b1, w2, b2, pair_t, pair_e, pair_first, pair_valid,
             offsets, ts=512):
    N, D = xs.shape
    NE, EH, _ = w1.shape
    P = pair_t.shape[0]

    def body(pt, pe, pf, pv, off, x_ref, w1_ref, b1_ref, w2_ref, b2_ref, o_ref):
        i = pl.program_id(0)
        e = pe[i]
        h = lax.dot_general(x_ref[...].astype(jnp.bfloat16), w1_ref[0],
                            (((1,), (1,)), ((), ())),
                            preferred_element_type=F32)
        h = jnp.maximum(h + b1_ref[0], 0.0)
        o = lax.dot_general(h.astype(jnp.bfloat16), w2_ref[0],
                            (((1,), (1,)), ((), ())),
                            preferred_element_type=F32) + b2_ref[0]
        pos = pt[i] * ts + lax.broadcasted_iota(I32, (ts, 1), 0)
        msk = (pos >= off[e]) & (pos < off[e + 1]) & (pv[i] == 1)
        o = jnp.where(msk, o, 0.0)

        @pl.when(pf[i] == 1)
        def _():
            o_ref[...] = o

        @pl.when(pf[i] == 0)
        def _():
            o_ref[...] += o

    grid_spec = pltpu.PrefetchScalarGridSpec(
        num_scalar_prefetch=5,
        grid=(P,),
        in_specs=[
            pl.BlockSpec((ts, D), lambda i, pt, pe, pf, pv, off: (pt[i], 0)),
            pl.BlockSpec((1, EH, D), lambda i, pt, pe, pf, pv, off: (pe[i], 0, 0)),
            pl.BlockSpec((1, 1, EH), lambda i, pt, pe, pf, pv, off: (pe[i], 0, 0)),
            pl.BlockSpec((1, D, EH), lambda i, pt, pe, pf, pv, off: (pe[i], 0, 0)),
            pl.BlockSpec((1, 1, D), lambda i, pt, pe, pf, pv, off: (pe[i], 0, 0)),
        ],
        out_specs=pl.BlockSpec((ts, D), lambda i, pt, pe, pf, pv, off: (pt[i], 0)),
    )
    return pl.pallas_call(
        body,
        grid_spec=grid_spec,
        out_shape=jax.ShapeDtypeStruct((N, D), F32),
    )(pair_t, pair_e, pair_first, pair_valid, offsets, xs, w1,
      b1.reshape(NE, 1, EH), w2, b2.reshape(NE, 1, D))


def _head(h, f1w, f1b, f2w, f2b):
    B, L, D = h.shape
    NC = f2w.shape[0]

    def body(h_ref, w1_ref, b1_ref, w2_ref, b2_ref, o_ref):
        hv = h_ref[...]
        s = jnp.sum(hv, axis=2)
        msk = (s != 0.0).astype(F32)[..., None]
        pooled = jnp.sum(hv * msk, axis=1) / jnp.clip(
            jnp.sum(msk, axis=1), 1.0, None)
        f = lax.dot_general(pooled, w1_ref[...], (((1,), (1,)), ((), ())),
                            preferred_element_type=F32) + b1_ref[...]
        f = jnp.maximum(f, 0.0)
        o_ref[...] = lax.dot_general(f, w2_ref[...], (((1,), (1,)), ((), ())),
                                     preferred_element_type=F32) + b2_ref[...]

    return pl.pallas_call(
        body,
        in_specs=[
            pl.BlockSpec((B, L, D), lambda: (0, 0, 0)),
            pl.BlockSpec(f1w.shape, lambda: (0, 0)),
            pl.BlockSpec((1, f1b.shape[1]), lambda: (0, 0)),
            pl.BlockSpec(f2w.shape, lambda: (0, 0)),
            pl.BlockSpec((1, NC), lambda: (0, 0)),
        ],
        out_specs=pl.BlockSpec((B, NC), lambda: (0, 0)),
        out_shape=jax.ShapeDtypeStruct((B, NC), F32),
    )(h, f1w, f1b, f2w, f2b)


def _routing_pairs(counts, n_tok, ts, ne):
    T = n_tok // ts
    P = T + ne - 1
    offsets = jnp.concatenate([jnp.zeros((1,), I32),
                               jnp.cumsum(counts).astype(I32)])
    t = jnp.arange(T, dtype=I32)[:, None]
    e = jnp.arange(ne, dtype=I32)[None, :]
    lo = t * ts
    hi = lo + ts
    st = offsets[:-1][None, :]
    en = offsets[1:][None, :]
    active = (en > lo) & (st < hi) & (en > st)
    key = t * ne + e
    big = jnp.int32(1 << 24)
    sortk = jnp.where(active, key, key + big)
    flat = jnp.sort(sortk.reshape(-1))[:P]
    valid = flat < big
    fk = jnp.where(valid, flat, flat - big)
    last = jnp.max(jnp.where(valid, fk, -1))
    fk = jnp.where(valid, fk, last)
    pair_t = (fk // ne).astype(I32)
    pair_e = (fk % ne).astype(I32)
    prev_t = jnp.concatenate([jnp.full((1,), -1, I32), pair_t[:-1]])
    pair_first = (valid & (pair_t != prev_t)).astype(I32)
    return offsets, pair_t, pair_e, pair_first, valid.astype(I32)


def kernel(params, x):
    p = params
    B, L = x.shape
    emb = p['embedding']
    V, D = emb.shape
    N = B * L

    ids = x.reshape(-1).astype(I32)
    rows = _sc_gather(emb, ids)
    pos = p['pos_encoding'][0, :L, :]
    h = _add_pos(rows.reshape(B, L, D), pos)

    for lp in p['layers']:
        mp = lp['mamba']
        DI = mp['conv_W'].shape[0]
        DS = mp['A_log'].shape[1]
        DTR = mp['dt_proj_W'].shape[1]

        hf = h.reshape(N, D)
        xc, z = _mm_split(h, mp['in_proj_W'], DI)
        at = -jnp.exp(mp['A_log']).T
        yz = _mamba_core(xc, z, mp['conv_W'].T, mp['conv_b'].reshape(1, DI),
                         mp['x_proj_W'], mp['dt_proj_W'],
                         mp['dt_proj_b'].reshape(1, DI), at,
                         mp['D'].reshape(1, DI), DTR, DS)
        h1 = _mm_ln_res(yz.reshape(N, DI), mp['out_proj_W'],
                        lp['ln1_g'].reshape(1, D), lp['ln1_b'].reshape(1, D),
                        hf)

        mo = lp['moe']
        NE = mo['gate_W'].shape[0]
        tw, ti, counts = _gate_top1(h1, mo['gate_W'], mo['gate_b'].reshape(1, NE),
                                    NE)
        ti_flat = ti.reshape(-1)
        sort_idx = jnp.argsort(ti_flat, stable=True).astype(I32)
        inv_perm = jnp.argsort(sort_idx).astype(I32)
        TS = 512
        offsets, pair_t, pair_e, pair_first, pair_valid = _routing_pairs(
            counts.reshape(-1), N, TS, NE)
        xs = _sc_gather(h1, sort_idx)
        os_ = _moe_ffn(xs, mo['W1'].astype(jnp.bfloat16), mo['b1'],
                       mo['W2'].astype(jnp.bfloat16), mo['b2'],
                       pair_t, pair_e, pair_first, pair_valid, offsets, ts=TS)
        moe_out = _sc_gather(os_, inv_perm)
        h2 = _scale_ln_res(moe_out, tw, lp['ln2_g'].reshape(1, D),
                           lp['ln2_b'].reshape(1, D), h1)
        h = h2.reshape(B, L, D)

    return _head(h, p['fc1_W'], p['fc1_b'].reshape(1, -1),
                 p['fc2_W'], p['fc2_b'].reshape(1, -1))


# speedup vs baseline: 1.0655x; 1.0655x over previous
import functools
import math

import jax
import jax.numpy as jnp
from jax import lax
from jax.experimental import pallas as pl
from jax.experimental.pallas import tpu as pltpu
from jax.experimental.pallas import tpu_sc as plsc

F32 = jnp.float32
I32 = jnp.int32


def _silu(x):
    return x * (0.5 * jnp.tanh(0.5 * x) + 0.5)


def _sc_gather(table, idx):
    V, D = table.shape
    N = idx.shape[0]
    info = plsc.get_sparse_core_info()
    nw = info.num_cores * info.num_subcores
    b_per_w = N // nw
    mesh = plsc.VectorSubcoreMesh(core_axis_name="c", subcore_axis_name="s")

    @functools.partial(
        pl.kernel,
        out_type=jax.ShapeDtypeStruct((N, D), F32),
        mesh=mesh,
        scratch_types=[
            pltpu.VMEM((b_per_w,), I32),
            pltpu.VMEM((b_per_w, D), F32),
            pltpu.SemaphoreType.DMA,
        ],
    )
    def k(table_hbm, idx_hbm, out_hbm, idx_v, rows_v, sem):
        wid = lax.axis_index("s") * info.num_cores + lax.axis_index("c")
        base = wid * b_per_w
        pltpu.sync_copy(idx_hbm.at[pl.ds(base, b_per_w)], idx_v)
        pltpu.async_copy(table_hbm.at[idx_v], rows_v, sem).wait()
        pltpu.sync_copy(rows_v, out_hbm.at[pl.ds(base, b_per_w)])

    return k(table, idx)


def _add_pos(emb, pos):
    B, L, D = emb.shape

    def body(e_ref, p_ref, o_ref):
        o_ref[...] = e_ref[...] + p_ref[...][None]

    return pl.pallas_call(
        body,
        grid=(B,),
        in_specs=[
            pl.BlockSpec((1, L, D), lambda b: (b, 0, 0)),
            pl.BlockSpec((L, D), lambda b: (0, 0)),
        ],
        out_specs=pl.BlockSpec((1, L, D), lambda b: (b, 0, 0)),
        out_shape=jax.ShapeDtypeStruct((B, L, D), F32),
    )(emb, pos)


def _mm_split(h, w, di, tm=512):
    B, L, D = h.shape
    M = w.shape[0]

    def body(x_ref, w_ref, xc_ref, z_ref):
        o = lax.dot_general(x_ref[0], w_ref[...], (((1,), (1,)), ((), ())),
                            preferred_element_type=F32)
        xc_ref[0] = o[:, :di]
        z_ref[0] = o[:, di:]

    return pl.pallas_call(
        body,
        grid=(B, L // tm),
        in_specs=[
            pl.BlockSpec((1, tm, D), lambda b, i: (b, i, 0)),
            pl.BlockSpec((M, D), lambda b, i: (0, 0)),
        ],
        out_specs=[
            pl.BlockSpec((1, tm, di), lambda b, i: (b, i, 0)),
            pl.BlockSpec((1, tm, di), lambda b, i: (b, i, 0)),
        ],
        out_shape=[
            jax.ShapeDtypeStruct((B, L, di), F32),
            jax.ShapeDtypeStruct((B, L, di), F32),
        ],
    )(h, w)


def _mamba_core(xc, z, wt, cb, xw, dtw, dtb, at, dv, dtr, ds, ch=512):
    B, L, DI = xc.shape
    DC = wt.shape[0]

    def body(x_ref, z_ref, wt_ref, cb_ref, xw_ref, dtw_ref, dtb_ref, at_ref,
             dv_ref, o_ref, h_ref, tail_ref, sdt_ref, sdtx_ref, sbm_ref,
             scm_ref):
        c = pl.program_id(1)

        @pl.when(c == 0)
        def _():
            h_ref[...] = jnp.zeros_like(h_ref)
            tail_ref[...] = jnp.zeros_like(tail_ref)

        x = x_ref[0]
        tail = tail_ref[...]
        xp = jnp.concatenate([tail[8 - (DC - 1):], x], axis=0)
        acc = cb_ref[...]
        for k in range(DC):
            acc = acc + xp[k:k + ch] * wt_ref[k][None]
        xcv = _silu(acc)
        tail_ref[...] = x[ch - 8:]

        xdbl = lax.dot_general(xcv, xw_ref[...], (((1,), (1,)), ((), ())),
                               preferred_element_type=F32)
        dtp = lax.dot_general(xdbl[:, :dtr], dtw_ref[...],
                              (((1,), (1,)), ((), ())),
                              preferred_element_type=F32) + dtb_ref[...]
        dt = jnp.maximum(dtp, 0.0) + jnp.log(1.0 + jnp.exp(-jnp.abs(dtp)))
        sdt_ref[...] = dt
        sdtx_ref[...] = dt * xcv
        sbm_ref[...] = xdbl[:, dtr:dtr + ds]
        scm_ref[...] = xdbl[:, dtr + ds:dtr + 2 * ds]

        atv = at_ref[...]

        def step(t, h):
            dtt = sdt_ref[pl.ds(t, 1), :]
            da = jnp.exp(atv * dtt)
            btc = sbm_ref[pl.ds(t, 1), :][0][:, None]
            dtxt = sdtx_ref[pl.ds(t, 1), :]
            h = da * h + btc * dtxt
            ctc = scm_ref[pl.ds(t, 1), :][0][:, None]
            o_ref[0, pl.ds(t, 1), :] = jnp.sum(h * ctc, axis=0, keepdims=True)
            return h

        h_ref[...] = lax.fori_loop(0, ch, step, h_ref[...], unroll=8)
        o_ref[0] = (o_ref[0] + dv_ref[...] * xcv) * _silu(z_ref[0])

    nxp = xw.shape[0]
    return pl.pallas_call(
        body,
        grid=(B, L // ch),
        in_specs=[
            pl.BlockSpec((1, ch, DI), lambda b, c: (b, c, 0)),
            pl.BlockSpec((1, ch, DI), lambda b, c: (b, c, 0)),
            pl.BlockSpec((DC, DI), lambda b, c: (0, 0)),
            pl.BlockSpec((1, DI), lambda b, c: (0, 0)),
            pl.BlockSpec((nxp, DI), lambda b, c: (0, 0)),
            pl.BlockSpec((DI, dtr), lambda b, c: (0, 0)),
            pl.BlockSpec((1, DI), lambda b, c: (0, 0)),
            pl.BlockSpec((ds, DI), lambda b, c: (0, 0)),
            pl.BlockSpec((1, DI), lambda b, c: (0, 0)),
        ],
        out_specs=pl.BlockSpec((1, ch, DI), lambda b, c: (b, c, 0)),
        out_shape=jax.ShapeDtypeStruct((B, L, DI), F32),
        scratch_shapes=[
            pltpu.VMEM((ds, DI), F32),
            pltpu.VMEM((8, DI), F32),
            pltpu.VMEM((ch, DI), F32),
            pltpu.VMEM((ch, DI), F32),
            pltpu.VMEM((ch, ds), F32),
            pltpu.VMEM((ch, ds), F32),
        ],
    )(xc, z, wt, cb, xw, dtw, dtb, at, dv)


def _mm_ln_res(x, w, g, b, res, tm=512, eps=1e-5):
    N, K = x.shape
    M = w.shape[0]

    def body(x_ref, w_ref, g_ref, b_ref, r_ref, o_ref):
        o = lax.dot_general(x_ref[...], w_ref[...], (((1,), (1,)), ((), ())),
                            preferred_element_type=F32)
        mu = jnp.mean(o, axis=1, keepdims=True)
        d = o - mu
        var = jnp.mean(d * d, axis=1, keepdims=True)
        ln = d * lax.rsqrt(var + eps) * g_ref[...] + b_ref[...]
        o_ref[...] = r_ref[...] + ln

    return pl.pallas_call(
        body,
        grid=(N // tm,),
        in_specs=[
            pl.BlockSpec((tm, K), lambda i: (i, 0)),
            pl.BlockSpec((M, K), lambda i: (0, 0)),
            pl.BlockSpec((1, M), lambda i: (0, 0)),
            pl.BlockSpec((1, M), lambda i: (0, 0)),
            pl.BlockSpec((tm, M), lambda i: (i, 0)),
        ],
        out_specs=pl.BlockSpec((tm, M), lambda i: (i, 0)),
        out_shape=jax.ShapeDtypeStruct((N, M), F32),
    )(x, w, g, b, res)


def _scale_ln_res(x, tw, g, b, res, tm=512, eps=1e-5):
    N, M = x.shape

    def body(x_ref, t_ref, g_ref, b_ref, r_ref, o_ref):
        v = x_ref[...] * t_ref[...]
        mu = jnp.mean(v, axis=1, keepdims=True)
        d = v - mu
        var = jnp.mean(d * d, axis=1, keepdims=True)
        ln = d * lax.rsqrt(var + eps) * g_ref[...] + b_ref[...]
        o_ref[...] = r_ref[...] + ln

    return pl.pallas_call(
        body,
        grid=(N // tm,),
        in_specs=[
            pl.BlockSpec((tm, M), lambda i: (i, 0)),
            pl.BlockSpec((tm, 1), lambda i: (i, 0)),
            pl.BlockSpec((1, M), lambda i: (0, 0)),
            pl.BlockSpec((1, M), lambda i: (0, 0)),
            pl.BlockSpec((tm, M), lambda i: (i, 0)),
        ],
        out_specs=pl.BlockSpec((tm, M), lambda i: (i, 0)),
        out_shape=jax.ShapeDtypeStruct((N, M), F32),
    )(x, tw, g, b, res)


def _gate_top1(x, gw, gb, ne, tm=512):
    N, K = x.shape

    def body(x_ref, w_ref, b_ref, tw_ref, ti_ref, cnt_ref):
        i = pl.program_id(0)
        logits = lax.dot_general(x_ref[...], w_ref[...], (((1,), (1,)), ((), ())),
                                 preferred_element_type=F32) + b_ref[...]
        m = jnp.max(logits, axis=1, keepdims=True)
        e = jnp.exp(logits - m)
        s = jnp.sum(e, axis=1, keepdims=True)
        gwm = e / s
        top = jnp.max(gwm, axis=1, keepdims=True)
        lanes = lax.broadcasted_iota(I32, (tm, ne), 1)
        idx = jnp.min(jnp.where(gwm >= top, lanes, ne), axis=1, keepdims=True)
        tw_ref[...] = top
        ti_ref[...] = idx

        @pl.when(i == 0)
        def _():
            cnt_ref[...] = jnp.zeros_like(cnt_ref)

        eq = (idx == lax.broadcasted_iota(I32, (tm, ne), 1)).astype(I32)
        cnt_ref[...] += jnp.sum(eq, axis=0, keepdims=True)

    return pl.pallas_call(
        body,
        grid=(N // tm,),
        in_specs=[
            pl.BlockSpec((tm, K), lambda i: (i, 0)),
            pl.BlockSpec((ne, K), lambda i: (0, 0)),
            pl.BlockSpec((1, ne), lambda i: (0, 0)),
        ],
        out_specs=[
            pl.BlockSpec((tm, 1), lambda i: (i, 0)),
            pl.BlockSpec((tm, 1), lambda i: (i, 0)),
            pl.BlockSpec((1, ne), lambda i: (0, 0)),
        ],
        out_shape=[
            jax.ShapeDtypeStruct((N, 1), F32),
            jax.ShapeDtypeStruct((N, 1), I32),
            jax.ShapeDtypeStruct((1, ne), I32),
        ],
    )(x, gw, gb)


def _moe_ffn(xs, w1, b1, w2, b2, pair_t, pair_e, pair_first, pair_valid,
             offsets, ts=512):
    N, D = xs.shape
    NE, EH, _ = w1.shape
    P = pair_t.shape[0]

    def body(pt, pe, pf, pv, off, x_ref, w1_ref, b1_ref, w2_ref, b2_ref, o_ref):
        i = pl.program_id(0)
        e = pe[i]
        h = lax.dot_general(x_ref[...], w1_ref[0], (((1,), (1,)), ((), ())),
                            preferred_element_type=F32)
        h = jnp.maximum(h + b1_ref[0], 0.0)
        o = lax.dot_general(h, w2_ref[0], (((1,), (1,)), ((), ())),
                            preferred_element_type=F32) + b2_ref[0]
        pos = pt[i] * ts + lax.broadcasted_iota(I32, (ts, 1), 0)
        msk = (pos >= off[e]) & (pos < off[e + 1]) & (pv[i] == 1)
        o = jnp.where(msk, o, 0.0)

        @pl.when(pf[i] == 1)
        def _():
            o_ref[...] = o

        @pl.when(pf[i] == 0)
        def _():
            o_ref[...] += o

    grid_spec = pltpu.PrefetchScalarGridSpec(
        num_scalar_prefetch=5,
        grid=(P,),
        in_specs=[
            pl.BlockSpec((ts, D), lambda i, pt, pe, pf, pv, off: (pt[i], 0)),
            pl.BlockSpec((1, EH, D), lambda i, pt, pe, pf, pv, off: (pe[i], 0, 0)),
            pl.BlockSpec((1, 1, EH), lambda i, pt, pe, pf, pv, off: (pe[i], 0, 0)),
            pl.BlockSpec((1, D, EH), lambda i, pt, pe, pf, pv, off: (pe[i], 0, 0)),
            pl.BlockSpec((1, 1, D), lambda i, pt, pe, pf, pv, off: (pe[i], 0, 0)),
        ],
        out_specs=pl.BlockSpec((ts, D), lambda i, pt, pe, pf, pv, off: (pt[i], 0)),
    )
    return pl.pallas_call(
        body,
        grid_spec=grid_spec,
        out_shape=jax.ShapeDtypeStruct((N, D), F32),
    )(pair_t, pair_e, pair_first, pair_valid, offsets, xs, w1,
      b1.reshape(NE, 1, EH), w2, b2.reshape(NE, 1, D))


def _head(h, f1w, f1b, f2w, f2b):
    B, L, D = h.shape
    NC = f2w.shape[0]

    def body(h_ref, w1_ref, b1_ref, w2_ref, b2_ref, o_ref):
        hv = h_ref[...]
        s = jnp.sum(hv, axis=2)
        msk = (s != 0.0).astype(F32)[..., None]
        pooled = jnp.sum(hv * msk, axis=1) / jnp.clip(
            jnp.sum(msk, axis=1), 1.0, None)
        f = lax.dot_general(pooled, w1_ref[...], (((1,), (1,)), ((), ())),
                            preferred_element_type=F32) + b1_ref[...]
        f = jnp.maximum(f, 0.0)
        o_ref[...] = lax.dot_general(f, w2_ref[...], (((1,), (1,)), ((), ())),
                                     preferred_element_type=F32) + b2_ref[...]

    return pl.pallas_call(
        body,
        in_specs=[
            pl.BlockSpec((B, L, D), lambda: (0, 0, 0)),
            pl.BlockSpec(f1w.shape, lambda: (0, 0)),
            pl.BlockSpec((1, f1b.shape[1]), lambda: (0, 0)),
            pl.BlockSpec(f2w.shape, lambda: (0, 0)),
            pl.BlockSpec((1, NC), lambda: (0, 0)),
        ],
        out_specs=pl.BlockSpec((B, NC), lambda: (0, 0)),
        out_shape=jax.ShapeDtypeStruct((B, NC), F32),
    )(h, f1w, f1b, f2w, f2b)


def _routing_pairs(counts, n_tok, ts, ne):
    T = n_tok // ts
    P = T + ne - 1
    offsets = jnp.concatenate([jnp.zeros((1,), I32),
                               jnp.cumsum(counts).astype(I32)])
    t = jnp.arange(T, dtype=I32)[:, None]
    e = jnp.arange(ne, dtype=I32)[None, :]
    lo = t * ts
    hi = lo + ts
    st = offsets[:-1][None, :]
    en = offsets[1:][None, :]
    active = (en > lo) & (st < hi) & (en > st)
    key = t * ne + e
    big = jnp.int32(1 << 24)
    sortk = jnp.where(active, key, key + big)
    flat = jnp.sort(sortk.reshape(-1))[:P]
    valid = flat < big
    fk = jnp.where(valid, flat, flat - big)
    last = jnp.max(jnp.where(valid, fk, -1))
    fk = jnp.where(valid, fk, last)
    pair_t = (fk // ne).astype(I32)
    pair_e = (fk % ne).astype(I32)
    prev_t = jnp.concatenate([jnp.full((1,), -1, I32), pair_t[:-1]])
    pair_first = (valid & (pair_t != prev_t)).astype(I32)
    return offsets, pair_t, pair_e, pair_first, valid.astype(I32)


def kernel(params, x):
    p = params
    B, L = x.shape
    emb = p['embedding']
    V, D = emb.shape
    N = B * L

    ids = x.reshape(-1).astype(I32)
    rows = _sc_gather(emb, ids)
    pos = p['pos_encoding'][0, :L, :]
    h = _add_pos(rows.reshape(B, L, D), pos)

    for lp in p['layers']:
        mp = lp['mamba']
        DI = mp['conv_W'].shape[0]
        DS = mp['A_log'].shape[1]
        DTR = mp['dt_proj_W'].shape[1]

        hf = h.reshape(N, D)
        xc, z = _mm_split(h, mp['in_proj_W'], DI)
        at = -jnp.exp(mp['A_log']).T
        yz = _mamba_core(xc, z, mp['conv_W'].T, mp['conv_b'].reshape(1, DI),
                         mp['x_proj_W'], mp['dt_proj_W'],
                         mp['dt_proj_b'].reshape(1, DI), at,
                         mp['D'].reshape(1, DI), DTR, DS)
        h1 = _mm_ln_res(yz.reshape(N, DI), mp['out_proj_W'],
                        lp['ln1_g'].reshape(1, D), lp['ln1_b'].reshape(1, D),
                        hf)

        mo = lp['moe']
        NE = mo['gate_W'].shape[0]
        tw, ti, counts = _gate_top1(h1, mo['gate_W'], mo['gate_b'].reshape(1, NE),
                                    NE)
        ti_flat = ti.reshape(-1)
        sort_idx = jnp.argsort(ti_flat, stable=True).astype(I32)
        inv_perm = jnp.argsort(sort_idx).astype(I32)
        TS = 512
        offsets, pair_t, pair_e, pair_first, pair_valid = _routing_pairs(
            counts.reshape(-1), N, TS, NE)
        xs = _sc_gather(h1, sort_idx)
        os_ = _moe_ffn(xs, mo['W1'], mo['b1'], mo['W2'], mo['b2'],
                       pair_t, pair_e, pair_first, pair_valid, offsets, ts=TS)
        moe_out = _sc_gather(os_, inv_perm)
        h2 = _scale_ln_res(moe_out, tw, lp['ln2_g'].reshape(1, D),
                           lp['ln2_b'].reshape(1, D), h1)
        h = h2.reshape(B, L, D)

    return _head(h, p['fc1_W'], p['fc1_b'].reshape(1, -1),
                 p['fc2_W'], p['fc2_b'].reshape(1, -1))
